# baseline (speedup 1.0000x reference)
import functools

import jax
import jax.numpy as jnp
from jax import lax
from jax.experimental import pallas as pl
from jax.experimental.pallas import tpu as pltpu
from jax.experimental.pallas import tpu_sc as plsc

B, S, D, V = 4096, 50, 300, 100000
DP = 304
NVEC = DP // 16
NC, NS = 2, 16
NW = NC * NS
N = B * S
PER_W = N // NW
CHUNK = 128
NCHUNK = PER_W // CHUNK
PACK = CHUNK * D

_mesh = plsc.VectorSubcoreMesh(core_axis_name="c", subcore_axis_name="s")


@functools.partial(
    pl.kernel,
    mesh=_mesh,
    out_type=jax.ShapeDtypeStruct((N * D,), jnp.float32),
    scratch_types=[
        pltpu.VMEM((NCHUNK, CHUNK), jnp.int32),
        pltpu.VMEM((CHUNK, DP), jnp.float32),
        pltpu.VMEM((PACK + 8,), jnp.float32),
        pltpu.SemaphoreType.DMA,
    ],
    compiler_params=pltpu.CompilerParams(use_tc_tiling_on_sc=False),
)
def _gather_kernel(idx_hbm, table_hbm, out_hbm, idx_v, rows_v, pack_v, sem):
    wid = lax.axis_index("s") * NC + lax.axis_index("c")
    pltpu.sync_copy(idx_hbm.at[wid], idx_v)
    base = wid * PER_W

    def chunk_body(c, carry):
        pltpu.async_copy(table_hbm.at[idx_v.at[c]], rows_v, sem).wait()

        def row_body(r, carry2):
            dst = r * D
            for v in range(NVEC):
                vec = rows_v[r, pl.ds(v * 16, 16)]
                pack_v[pl.ds(dst + v * 16, 16)] = vec
            return carry2

        lax.fori_loop(0, CHUNK, row_body, 0)
        pltpu.sync_copy(pack_v.at[pl.ds(0, PACK)],
                        out_hbm.at[pl.ds((base + c * CHUNK) * D, PACK)])
        return carry

    lax.fori_loop(0, NCHUNK, chunk_body, 0)


def kernel(indices, table):
    idx = indices.astype(jnp.int32).reshape(NW, NCHUNK, CHUNK)
    table_p = jnp.pad(table, ((0, 0), (0, DP - D)))
    out = _gather_kernel(idx, table_p)
    return out.reshape(B, S, D)


# speedup vs baseline: 1.5799x; 1.5799x over previous
import functools

import jax
import jax.numpy as jnp
from jax import lax
from jax.experimental import pallas as pl
from jax.experimental.pallas import tpu as pltpu
from jax.experimental.pallas import tpu_sc as plsc

B, S, D, V = 4096, 50, 300, 100000
DP = 384
NC, NS = 2, 16
NW = NC * NS
N = B * S
PER_W = N // NW
CHUNK = 128
NCHUNK = PER_W // CHUNK

_mesh = plsc.VectorSubcoreMesh(core_axis_name="c", subcore_axis_name="s")


@functools.partial(
    pl.kernel,
    mesh=_mesh,
    out_type=jax.ShapeDtypeStruct((N, DP), jnp.float32),
    scratch_types=[
        pltpu.VMEM((NCHUNK, CHUNK), jnp.int32),
        pltpu.VMEM((CHUNK, DP), jnp.float32),
        pltpu.SemaphoreType.DMA,
    ],
    compiler_params=pltpu.CompilerParams(use_tc_tiling_on_sc=True),
)
def _gather_kernel(idx_hbm, table_hbm, out_hbm, idx_v, rows_v, sem):
    wid = lax.axis_index("s") * NC + lax.axis_index("c")
    pltpu.sync_copy(idx_hbm.at[wid], idx_v)
    base = wid * PER_W

    def chunk_body(c, carry):
        pltpu.async_copy(table_hbm.at[idx_v.at[c]], rows_v, sem).wait()
        pltpu.sync_copy(rows_v, out_hbm.at[pl.ds(base + c * CHUNK, CHUNK)])
        return carry

    lax.fori_loop(0, NCHUNK, chunk_body, 0)


def kernel(indices, table):
    idx = indices.astype(jnp.int32).reshape(NW, NCHUNK, CHUNK)
    table_p = jnp.pad(table, ((0, 0), (0, DP - D)))
    out = _gather_kernel(idx, table_p)
    return out[:, :D].reshape(B, S, D)


# speedup vs baseline: 1.7485x; 1.1068x over previous
import functools

import jax
import jax.numpy as jnp
from jax import lax
from jax.experimental import pallas as pl
from jax.experimental.pallas import tpu as pltpu
from jax.experimental.pallas import tpu_sc as plsc

B, S, D, V = 4096, 50, 300, 100000
DP = 384
NC, NS = 2, 16
NW = NC * NS
N = B * S
PER_W = N // NW
CHUNK = 128
NCHUNK = PER_W // CHUNK

_mesh = plsc.VectorSubcoreMesh(core_axis_name="c", subcore_axis_name="s")


@functools.partial(
    pl.kernel,
    mesh=_mesh,
    out_type=jax.ShapeDtypeStruct((N, DP), jnp.float32),
    scratch_types=[
        pltpu.VMEM((NCHUNK, CHUNK), jnp.int32),
        pltpu.VMEM((CHUNK, DP), jnp.float32),
        pltpu.SemaphoreType.DMA,
    ],
    compiler_params=pltpu.CompilerParams(use_tc_tiling_on_sc=True),
)
def _gather_kernel(idx_hbm, table_hbm, out_hbm, idx_v, rows_v, sem):
    wid = lax.axis_index("s") * NC + lax.axis_index("c")
    pltpu.sync_copy(idx_hbm.at[wid], idx_v)
    base = wid * PER_W

    def chunk_body(c, carry):
        pltpu.async_copy(table_hbm.at[idx_v.at[c]], rows_v, sem).wait()
        pltpu.sync_copy(rows_v, out_hbm.at[pl.ds(base + c * CHUNK, CHUNK)])
        return carry

    lax.fori_loop(0, NCHUNK, chunk_body, 0)


_PAD_ROWS = 800


def _pad_body(in_ref, out_ref):
    out_ref[:, :D] = in_ref[...]


_pad_tc = pl.pallas_call(
    _pad_body,
    grid=(V // _PAD_ROWS,),
    in_specs=[pl.BlockSpec((_PAD_ROWS, D), lambda i: (i, 0))],
    out_specs=pl.BlockSpec((_PAD_ROWS, DP), lambda i: (i, 0)),
    out_shape=jax.ShapeDtypeStruct((V, DP), jnp.float32),
)

_KB = 8


def _depad_body(in_ref, out_ref):
    for k in range(_KB):
        out_ref[k] = in_ref[pl.ds(k * S, S), pl.ds(0, D)]


_depad_tc = pl.pallas_call(
    _depad_body,
    grid=(B // _KB,),
    in_specs=[pl.BlockSpec((_KB * S, DP), lambda i: (i, 0))],
    out_specs=pl.BlockSpec((_KB, S, D), lambda i: (i, 0, 0)),
    out_shape=jax.ShapeDtypeStruct((B, S, D), jnp.float32),
)


def kernel(indices, table):
    idx = indices.astype(jnp.int32).reshape(NW, NCHUNK, CHUNK)
    table_p = _pad_tc(table)
    out = _gather_kernel(idx, table_p)
    return _depad_tc(out)


# speedup vs baseline: 1.8749x; 1.0723x over previous
import functools

import jax
import jax.numpy as jnp
from jax import lax
from jax.experimental import pallas as pl
from jax.experimental.pallas import tpu as pltpu
from jax.experimental.pallas import tpu_sc as plsc

B, S, D, V = 4096, 50, 300, 100000
DP = 384
NC, NS = 2, 16
NW = NC * NS
N = B * S
PER_W = N // NW
CHUNK = 128
NCHUNK = PER_W // CHUNK

_mesh = plsc.VectorSubcoreMesh(core_axis_name="c", subcore_axis_name="s")


@functools.partial(
    pl.kernel,
    mesh=_mesh,
    out_type=jax.ShapeDtypeStruct((N, DP), jnp.float32),
    scratch_types=[
        pltpu.VMEM((NCHUNK, CHUNK), jnp.int32),
        pltpu.VMEM((CHUNK, DP), jnp.float32),
        pltpu.SemaphoreType.DMA,
    ],
    compiler_params=pltpu.CompilerParams(use_tc_tiling_on_sc=True),
)
def _gather_kernel(idx_hbm, table_hbm, out_hbm, idx_v, rows_v, sem):
    wid = lax.axis_index("s") * NC + lax.axis_index("c")
    pltpu.sync_copy(idx_hbm.at[wid], idx_v)
    base = wid * PER_W

    def chunk_body(c, carry):
        pltpu.async_copy(table_hbm.at[idx_v.at[c]], rows_v, sem).wait()
        pltpu.sync_copy(rows_v, out_hbm.at[pl.ds(base + c * CHUNK, CHUNK)])
        return carry

    lax.fori_loop(0, NCHUNK, chunk_body, 0)


_PAD_ROWS = 800


def _pad_body(in_ref, out_ref):
    out_ref[:, :D] = in_ref[...]


_pad_tc = pl.pallas_call(
    _pad_body,
    grid=(V // _PAD_ROWS,),
    in_specs=[pl.BlockSpec((_PAD_ROWS, D), lambda i: (i, 0))],
    out_specs=pl.BlockSpec((_PAD_ROWS, DP), lambda i: (i, 0)),
    out_shape=jax.ShapeDtypeStruct((V, DP), jnp.float32),
)

_KB = 8


def _depad_body(in_ref, out_ref):
    for k in range(_KB):
        out_ref[k] = in_ref[pl.ds(k * S, S), pl.ds(0, D)]


_depad_tc = pl.pallas_call(
    _depad_body,
    grid=(B // _KB,),
    in_specs=[pl.BlockSpec((_KB * S, DP), lambda i: (i, 0))],
    out_specs=pl.BlockSpec((_KB, S, D), lambda i: (i, 0, 0)),
    out_shape=jax.ShapeDtypeStruct((B, S, D), jnp.float32),
)


def kernel(indices, table):
    idx = indices.astype(jnp.int32).reshape(NW, NCHUNK, CHUNK)
    table_p = _pad_tc(table)
    out = _gather_kernel(idx, table_p)
    return jax.lax.optimization_barrier(_depad_tc(out))


# speedup vs baseline: 2.1285x; 1.1352x over previous
import functools

import jax
import jax.numpy as jnp
from jax import lax
from jax.experimental import pallas as pl
from jax.experimental.pallas import tpu as pltpu
from jax.experimental.pallas import tpu_sc as plsc

B, S, D, V = 4096, 50, 300, 100000
DP = 384
NC, NS = 2, 16
NW = NC * NS
N = B * S
PER_W = N // NW
CHUNK = 128
NCHUNK = PER_W // CHUNK

_mesh = plsc.VectorSubcoreMesh(core_axis_name="c", subcore_axis_name="s")


@functools.partial(
    pl.kernel,
    mesh=_mesh,
    out_type=jax.ShapeDtypeStruct((N, DP), jnp.float32),
    scratch_types=[
        pltpu.VMEM((NCHUNK, CHUNK), jnp.int32),
        pltpu.VMEM((CHUNK, DP), jnp.float32),
        pltpu.SemaphoreType.DMA,
    ],
    compiler_params=pltpu.CompilerParams(use_tc_tiling_on_sc=True),
)
def _gather_kernel(idx_hbm, table_hbm, out_hbm, idx_v, rows_v, sem):
    wid = lax.axis_index("s") * NC + lax.axis_index("c")
    pltpu.sync_copy(idx_hbm.at[wid], idx_v)
    base = wid * PER_W

    def chunk_body(c, carry):
        pltpu.async_copy(table_hbm.at[idx_v.at[c]], rows_v, sem).wait()
        pltpu.sync_copy(rows_v, out_hbm.at[pl.ds(base + c * CHUNK, CHUNK)])
        return carry

    lax.fori_loop(0, NCHUNK, chunk_body, 0)


_PAD_ROWS = 1024


def _pad_body(in_ref, out_ref):
    out_ref[:, :D] = in_ref[...].T


_pad_tc = pl.pallas_call(
    _pad_body,
    grid=((V + _PAD_ROWS - 1) // _PAD_ROWS,),
    in_specs=[pl.BlockSpec((D, _PAD_ROWS), lambda i: (0, i))],
    out_specs=pl.BlockSpec((_PAD_ROWS, DP), lambda i: (i, 0)),
    out_shape=jax.ShapeDtypeStruct((V, DP), jnp.float32),
)

_KB = 8


def _depad_body(in_ref, out_ref):
    for k in range(_KB):
        out_ref[k] = in_ref[pl.ds(k * S, S), pl.ds(0, D)]


_depad_tc = pl.pallas_call(
    _depad_body,
    grid=(B // _KB,),
    in_specs=[pl.BlockSpec((_KB * S, DP), lambda i: (i, 0))],
    out_specs=pl.BlockSpec((_KB, S, D), lambda i: (i, 0, 0)),
    out_shape=jax.ShapeDtypeStruct((B, S, D), jnp.float32),
)


def kernel(indices, table):
    idx = indices.astype(jnp.int32).reshape(NW, NCHUNK, CHUNK)
    table_p = _pad_tc(table.T)
    out = _gather_kernel(idx, table_p)
    return jax.lax.optimization_barrier(_depad_tc(out))


# speedup vs baseline: 2.2392x; 1.0520x over previous
import functools

import jax
import jax.numpy as jnp
from jax import lax
from jax.experimental import pallas as pl
from jax.experimental.pallas import tpu as pltpu
from jax.experimental.pallas import tpu_sc as plsc

B, S, D, V = 4096, 50, 300, 100000
DP = 384
NC, NS = 2, 16
NW = NC * NS
N = B * S
CHUNK = 128
NSL = 2
N_SL = N // NSL
PER_W = N_SL // NW
NCHUNK = PER_W // CHUNK

_mesh = plsc.VectorSubcoreMesh(core_axis_name="c", subcore_axis_name="s")


@functools.partial(
    pl.kernel,
    mesh=_mesh,
    out_type=jax.ShapeDtypeStruct((N_SL, DP), jnp.float32),
    scratch_types=[
        pltpu.VMEM((NCHUNK, CHUNK), jnp.int32),
        pltpu.VMEM((CHUNK, DP), jnp.float32),
        pltpu.SemaphoreType.DMA,
    ],
    compiler_params=pltpu.CompilerParams(use_tc_tiling_on_sc=True),
)
def _gather_slice(idx_hbm, table_hbm, out_hbm, idx_v, rows_v, sem):
    wid = lax.axis_index("s") * NC + lax.axis_index("c")
    pltpu.sync_copy(idx_hbm.at[wid], idx_v)
    base = wid * PER_W

    def chunk_body(c, carry):
        pltpu.async_copy(table_hbm.at[idx_v.at[c]], rows_v, sem).wait()
        pltpu.sync_copy(rows_v, out_hbm.at[pl.ds(base + c * CHUNK, CHUNK)])
        return carry

    lax.fori_loop(0, NCHUNK, chunk_body, 0)


_PAD_ROWS = 1024


def _pad_body(in_ref, out_ref):
    out_ref[:, :D] = in_ref[...].T


_pad_tc = pl.pallas_call(
    _pad_body,
    grid=((V + _PAD_ROWS - 1) // _PAD_ROWS,),
    in_specs=[pl.BlockSpec((D, _PAD_ROWS), lambda i: (0, i))],
    out_specs=pl.BlockSpec((_PAD_ROWS, DP), lambda i: (i, 0)),
    out_shape=jax.ShapeDtypeStruct((V, DP), jnp.float32),
)

_KB = 8
_BLOCKS_SL = (B // NSL) // _KB


def _depad_first_body(in_ref, out_ref):
    for k in range(_KB):
        out_ref[k] = in_ref[pl.ds(k * S, S), pl.ds(0, D)]


def _depad_next_body(in_ref, prev_ref, out_ref):
    del prev_ref
    for k in range(_KB):
        out_ref[k] = in_ref[pl.ds(k * S, S), pl.ds(0, D)]


_depad_first = pl.pallas_call(
    _depad_first_body,
    grid=(_BLOCKS_SL,),
    in_specs=[pl.BlockSpec((_KB * S, DP), lambda i: (i, 0))],
    out_specs=pl.BlockSpec((_KB, S, D), lambda i: (i, 0, 0)),
    out_shape=jax.ShapeDtypeStruct((B, S, D), jnp.float32),
)


def _make_depad_next(sl):
    off = sl * _BLOCKS_SL
    return pl.pallas_call(
        _depad_next_body,
        grid=(_BLOCKS_SL,),
        in_specs=[
            pl.BlockSpec((_KB * S, DP), lambda i: (i, 0)),
            pl.BlockSpec(memory_space=pl.ANY),
        ],
        out_specs=pl.BlockSpec((_KB, S, D), lambda i, off=off: (i + off, 0, 0)),
        out_shape=jax.ShapeDtypeStruct((B, S, D), jnp.float32),
        input_output_aliases={1: 0},
    )


_depad_rest = [_make_depad_next(sl) for sl in range(1, NSL)]


def kernel(indices, table):
    idxf = indices.astype(jnp.int32).reshape(-1)
    table_p = _pad_tc(table.T)
    outs = []
    for sl in range(NSL):
        idx_sl = idxf[sl * N_SL:(sl + 1) * N_SL].reshape(NW, NCHUNK, CHUNK)
        outs.append(_gather_slice(idx_sl, table_p))
    final = _depad_first(outs[0])
    for sl in range(1, NSL):
        final = _depad_rest[sl - 1](outs[sl], final)
    return jax.lax.optimization_barrier(final)
